# baseline (speedup 1.0000x reference)
import functools
import jax
import jax.numpy as jnp
from jax import lax
from jax.experimental import pallas as pl
from jax.experimental.pallas import tpu as pltpu
from jax.experimental.pallas import tpu_sc as plsc

_TIMESTEPS = 1000
_MAGIC = 8388608.0
_B = 16384
_LANES = 16

_info = plsc.get_sparse_core_info()
_NC, _NS = 1, 8
_NW = _NC * _NS
_CHUNK = _B // _NW
_GAMMA_N = 1001


def _body(t_hbm, gamma_hbm, out_hbm, gamma_v, t_v, out_v, sem_g, sem_t):
    wid = lax.axis_index("s") * _NC + lax.axis_index("c")
    base = wid * _CHUNK
    cp_g = pltpu.async_copy(gamma_hbm, gamma_v, sem_g)
    cp_t = pltpu.async_copy(t_hbm.at[pl.ds(base, _CHUNK)], t_v, sem_t)
    cp_g.wait()
    cp_t.wait()

    def step(i, carry):
        off = i * _LANES
        tv = t_v[pl.ds(off, _LANES)]
        xf = (tv * float(_TIMESTEPS) + _MAGIC) - _MAGIC
        idx = xf.astype(jnp.int32)
        out_v[pl.ds(off, _LANES)] = plsc.load_gather(gamma_v, [idx])
        return carry

    lax.fori_loop(0, _CHUNK // _LANES, step, 0)
    pltpu.sync_copy(out_v, out_hbm.at[pl.ds(base, _CHUNK)])


_mesh = plsc.VectorSubcoreMesh(
    core_axis_name="c", subcore_axis_name="s", num_cores=_NC, num_subcores=_NS
)

_sc_lookup = pl.kernel(
    _body,
    out_type=jax.ShapeDtypeStruct((_B,), jnp.float32),
    mesh=_mesh,
    scratch_types=[
        pltpu.VMEM((_GAMMA_N,), jnp.float32),
        pltpu.VMEM((_CHUNK,), jnp.float32),
        pltpu.VMEM((_CHUNK,), jnp.float32),
        pltpu.SemaphoreType.DMA,
        pltpu.SemaphoreType.DMA,
    ],
    compiler_params=pltpu.CompilerParams(
        needs_layout_passes=False,
        skip_device_barrier=True,
        disable_bounds_checks=True,
        disable_semaphore_checks=True,
    ),
)


@jax.jit
def kernel(t, gamma):
    out = _sc_lookup(t.reshape(_B), gamma)
    return out.reshape(t.shape)


# speedup vs baseline: 1.0010x; 1.0010x over previous
import functools
import jax
import jax.numpy as jnp
from jax import lax
from jax.experimental import pallas as pl
from jax.experimental.pallas import tpu as pltpu
from jax.experimental.pallas import tpu_sc as plsc

_TIMESTEPS = 1000
_MAGIC = 8388608.0
_B = 16384
_LANES = 16

_info = plsc.get_sparse_core_info()
_NC, _NS = 1, _info.num_subcores
_NW = _NC * _NS
_CHUNK = _B // _NW
_GAMMA_N = 1001


def _body(t_hbm, gamma_hbm, out_hbm, gamma_v, t_v, out_v, sem_g, sem_t):
    wid = lax.axis_index("s") * _NC + lax.axis_index("c")
    base = wid * _CHUNK
    cp_g = pltpu.async_copy(gamma_hbm, gamma_v, sem_g)
    cp_t = pltpu.async_copy(t_hbm.at[pl.ds(base, _CHUNK)], t_v, sem_t)
    cp_g.wait()
    cp_t.wait()

    @plsc.parallel_loop(0, _CHUNK // _LANES, 1, unroll=4)
    def step(i):
        off = i * _LANES
        tv = t_v[pl.ds(off, _LANES)]
        xf = (tv * float(_TIMESTEPS) + _MAGIC) - _MAGIC
        idx = xf.astype(jnp.int32)
        out_v[pl.ds(off, _LANES)] = plsc.load_gather(gamma_v, [idx])

    pltpu.sync_copy(out_v, out_hbm.at[pl.ds(base, _CHUNK)])


_mesh = plsc.VectorSubcoreMesh(
    core_axis_name="c", subcore_axis_name="s", num_cores=_NC, num_subcores=_NS
)

_sc_lookup = pl.kernel(
    _body,
    out_type=jax.ShapeDtypeStruct((_B,), jnp.float32),
    mesh=_mesh,
    scratch_types=[
        pltpu.VMEM((_GAMMA_N,), jnp.float32),
        pltpu.VMEM((_CHUNK,), jnp.float32),
        pltpu.VMEM((_CHUNK,), jnp.float32),
        pltpu.SemaphoreType.DMA,
        pltpu.SemaphoreType.DMA,
    ],
    compiler_params=pltpu.CompilerParams(
        needs_layout_passes=False,
        skip_device_barrier=True,
        disable_bounds_checks=True,
        disable_semaphore_checks=True,
    ),
)


@jax.jit
def kernel(t, gamma):
    out = _sc_lookup(t.reshape(_B), gamma)
    return out.reshape(t.shape)
